# Initial kernel scaffold; baseline (speedup 1.0000x reference)
import jax
import jax.numpy as jnp
from jax.experimental import pallas as pl


def kernel(feat_full, edge_index, emb_table, W_self1, W_neigh1, b1, W_self2, W_neigh2, b2, W_cls, b_cls):
    raise NotImplementedError("write your pallas kernel here")


# speedup vs baseline: 2.4154x; 2.4154x over previous
import functools

import jax
import jax.numpy as jnp
from jax import lax
from jax.experimental import pallas as pl
from jax.experimental.pallas import tpu as pltpu
from jax.experimental.pallas import tpu_sc as plsc

N = 10000
L = 50
VOCAB = 30000
E = 160000
D = 128
C = 50

NC = 2
NS = 16
NW = NC * NS

N_PAD = 10240
NODES_PER_TILE = N_PAD // NW
LT = 56
E_PAD = 163840
EPT = E_PAD // NW
ECHUNK = 128
NECH = EPT // ECHUNK
STRIPE = N_PAD // NS

_SC_MESH = dict(core_axis_name="c", subcore_axis_name="s")


def _worker_id():
  return lax.axis_index("s") * NC + lax.axis_index("c")


def _maxpool_body(feat_ref, emb_ref, x_ref, idx_v, rv, out_v, sem0, sem1):
  w = _worker_id()
  node_base = w * NODES_PER_TILE
  pltpu.sync_copy(
      feat_ref.at[pl.ds(node_base * LT, NODES_PER_TILE * LT)], idx_v)

  sems = (sem0, sem1)
  n_chunks = NODES_PER_TILE // 2

  def gather_start(j, b):
    pltpu.make_async_copy(
        emb_ref.at[idx_v.at[pl.ds(j * (2 * LT), 2 * LT)]],
        rv.at[b], sems[b]).start()

  def gather_wait(b):
    pltpu.make_async_copy(
        emb_ref.at[idx_v.at[pl.ds(0, 2 * LT)]], rv.at[b], sems[b]).wait()

  gather_start(0, 0)
  gather_start(1, 1)

  @pl.loop(0, n_chunks, step=2)
  def _chunks(j):
    for b in range(2):
      jj = j + b
      gather_wait(b)
      for n in range(2):
        row0 = n * LT
        acc = tuple(rv[b, row0, pl.ds(k * 16, 16)] for k in range(8))

        def rbody(r, a, _b=b, _row0=row0):
          return tuple(
              jnp.maximum(a[k], rv[_b, _row0 + r, pl.ds(k * 16, 16)])
              for k in range(8))

        acc = lax.fori_loop(1, L, rbody, acc)
        for k in range(8):
          out_v[jj * 2 + n, pl.ds(k * 16, 16)] = acc[k]

      @pl.when(jj + 2 < n_chunks)
      def _refill(_jj=jj, _b=b):
        gather_start(_jj + 2, _b)

  pltpu.sync_copy(out_v, x_ref.at[pl.ds(node_base, NODES_PER_TILE)])


def _maxpool_call(feat_flat, emb_table):
  mesh = plsc.VectorSubcoreMesh(**_SC_MESH)
  f = pl.kernel(
      _maxpool_body,
      out_type=jax.ShapeDtypeStruct((N_PAD, D), jnp.float32),
      mesh=mesh,
      scratch_types=[
          pltpu.VMEM((NODES_PER_TILE * LT,), jnp.int32),
          pltpu.VMEM((2, 2 * LT, D), jnp.float32),
          pltpu.VMEM((NODES_PER_TILE, D), jnp.float32),
          pltpu.SemaphoreType.DMA,
          pltpu.SemaphoreType.DMA,
      ],
  )
  return f(feat_flat, emb_table)


def _agg_body(with_deg, z_ref, src_ref, dst_ref, t_ref, deg_ref,
              src_v, dst_v, rv, ones_v, zd_v, agg_sh, deg_sh, sem0, sem1):
  c = lax.axis_index("c")
  s = lax.axis_index("s")
  w = s * NC + c
  pltpu.sync_copy(src_ref.at[pl.ds(w * EPT, EPT)], src_v)
  pltpu.sync_copy(dst_ref.at[pl.ds(w * EPT, EPT)], dst_v)

  zeros16 = jnp.zeros((16,), jnp.float32)

  @pl.loop(0, ECHUNK)
  def _zrow(i):
    for k in range(8):
      rv[0, i, pl.ds(k * 16, 16)] = zeros16

  stripe0 = s * STRIPE
  for q in range(STRIPE // ECHUNK):
    pltpu.sync_copy(rv.at[0], agg_sh.at[pl.ds(stripe0 + q * ECHUNK, ECHUNK)])

  if with_deg:
    ones16 = jnp.ones((16,), jnp.float32)
    for k in range(8):
      ones_v[pl.ds(k * 16, 16)] = ones16

    @pl.loop(0, STRIPE // 16)
    def _zdeg(i):
      zd_v[pl.ds(i * 16, 16)] = zeros16

    pltpu.sync_copy(zd_v, deg_sh.at[pl.ds(stripe0, STRIPE)])

  plsc.subcore_barrier()

  sems = (sem0, sem1)

  def gather_start(j, b):
    pltpu.make_async_copy(
        z_ref.at[src_v.at[pl.ds(j * ECHUNK, ECHUNK)]], rv.at[b],
        sems[b]).start()

  def gather_wait(b):
    pltpu.make_async_copy(
        z_ref.at[src_v.at[pl.ds(0, ECHUNK)]], rv.at[b], sems[b]).wait()

  gather_start(0, 0)
  gather_start(1, 1)

  @pl.loop(0, NECH, step=2)
  def _chunks(j):
    for b in range(2):
      jj = j + b
      gather_wait(b)
      dsl = dst_v.at[pl.ds(jj * ECHUNK, ECHUNK)]
      pltpu.sync_copy(rv.at[b], agg_sh.at[dsl], add=True)
      if with_deg:
        pltpu.sync_copy(ones_v, deg_sh.at[dsl], add=True)

      @pl.when(jj + 2 < NECH)
      def _refill(_jj=jj, _b=b):
        gather_start(_jj + 2, _b)

  plsc.subcore_barrier()

  pltpu.sync_copy(agg_sh.at[pl.ds(stripe0, STRIPE)],
                  t_ref.at[c].at[pl.ds(stripe0, STRIPE)])
  if with_deg:
    pltpu.sync_copy(deg_sh.at[pl.ds(stripe0, STRIPE)],
                    deg_ref.at[c].at[pl.ds(stripe0, STRIPE)])


def _agg_call(z, src, dst, with_deg):
  mesh = plsc.VectorSubcoreMesh(**_SC_MESH)
  out_type = [jax.ShapeDtypeStruct((NC, N_PAD, D), jnp.float32)]
  if with_deg:
    out_type.append(jax.ShapeDtypeStruct((NC, N_PAD), jnp.float32))

  def body(z_ref, src_ref, dst_ref, *rest):
    if with_deg:
      t_ref, deg_ref = rest[0], rest[1]
      scr = rest[2:]
    else:
      t_ref, deg_ref = rest[0], None
      scr = rest[1:]
    _agg_body(with_deg, z_ref, src_ref, dst_ref, t_ref, deg_ref, *scr)

  f = pl.kernel(
      body,
      out_type=tuple(out_type),
      mesh=mesh,
      scratch_types=[
          pltpu.VMEM((EPT,), jnp.int32),
          pltpu.VMEM((EPT,), jnp.int32),
          pltpu.VMEM((2, ECHUNK, D), jnp.float32),
          pltpu.VMEM((ECHUNK,), jnp.float32),
          pltpu.VMEM((STRIPE,), jnp.float32),
          pltpu.VMEM_SHARED((N_PAD, D), jnp.float32),
          pltpu.VMEM_SHARED((N_PAD,), jnp.float32),
          pltpu.SemaphoreType.DMA,
          pltpu.SemaphoreType.DMA,
      ],
  )
  return f(z, src, dst)


_BLK = 1280


def _full(shape):
  return pl.BlockSpec(shape, lambda i: tuple(0 for _ in shape))


def _dense1_call(x, ws, wn, b):
  def body(x_ref, ws_ref, wn_ref, b_ref, hs_ref, z_ref):
    xb = x_ref[...]
    bb = b_ref[...]
    hs_ref[...] = jnp.maximum(
        jnp.dot(xb, ws_ref[...], preferred_element_type=jnp.float32) + bb, 0.0)
    z_ref[...] = jnp.dot(xb, wn_ref[...], preferred_element_type=jnp.float32)

  return pl.pallas_call(
      body,
      grid=(N_PAD // _BLK,),
      in_specs=[
          pl.BlockSpec((_BLK, D), lambda i: (i, 0)),
          _full((D, D)),
          _full((D, D)),
          _full((1, D)),
      ],
      out_specs=[
          pl.BlockSpec((_BLK, D), lambda i: (i, 0)),
          pl.BlockSpec((_BLK, D), lambda i: (i, 0)),
      ],
      out_shape=[
          jax.ShapeDtypeStruct((N_PAD, D), jnp.float32),
          jax.ShapeDtypeStruct((N_PAD, D), jnp.float32),
      ],
  )(x, ws, wn, b.reshape(1, D))


def _dense2_call(t1, deg, hs1, b1, ws2, wn2, b2):
  def body(t_ref, deg_ref, hs1_ref, b1_ref, ws2_ref, wn2_ref, b2_ref,
           hs2_ref, z2_ref):
    t = t_ref[0] + t_ref[1]
    dg = deg_ref[0] + deg_ref[1]
    dinv = 1.0 / jnp.maximum(dg, 1.0)
    hn1 = jnp.maximum(t * dinv + b1_ref[...], 0.0)
    hs1b = hs1_ref[...]
    hs2_ref[...] = jnp.maximum(
        jnp.dot(hs1b, ws2_ref[0:D], preferred_element_type=jnp.float32)
        + jnp.dot(hn1, ws2_ref[D:2 * D], preferred_element_type=jnp.float32)
        + b2_ref[...], 0.0)
    z2_ref[...] = (
        jnp.dot(hs1b, wn2_ref[0:D], preferred_element_type=jnp.float32)
        + jnp.dot(hn1, wn2_ref[D:2 * D], preferred_element_type=jnp.float32))

  return pl.pallas_call(
      body,
      grid=(N_PAD // _BLK,),
      in_specs=[
          pl.BlockSpec((NC, _BLK, D), lambda i: (0, i, 0)),
          pl.BlockSpec((NC, _BLK, 1), lambda i: (0, i, 0)),
          pl.BlockSpec((_BLK, D), lambda i: (i, 0)),
          _full((1, D)),
          _full((2 * D, D)),
          _full((2 * D, D)),
          _full((1, D)),
      ],
      out_specs=[
          pl.BlockSpec((_BLK, D), lambda i: (i, 0)),
          pl.BlockSpec((_BLK, D), lambda i: (i, 0)),
      ],
      out_shape=[
          jax.ShapeDtypeStruct((N_PAD, D), jnp.float32),
          jax.ShapeDtypeStruct((N_PAD, D), jnp.float32),
      ],
  )(t1, deg.reshape(NC, N_PAD, 1), hs1, b1.reshape(1, D), ws2, wn2,
    b2.reshape(1, D))


def _cls_call(t2, deg, hs2, x, b2, wc, bc):
  def body(t_ref, deg_ref, hs2_ref, x_ref, b2_ref, wc_ref, bc_ref, out_ref):
    t = t_ref[0] + t_ref[1]
    dg = deg_ref[0] + deg_ref[1]
    dinv = 1.0 / jnp.maximum(dg, 1.0)
    hn2 = jnp.maximum(t * dinv + b2_ref[...], 0.0)
    out_ref[...] = (
        jnp.dot(hs2_ref[...], wc_ref[0:D], preferred_element_type=jnp.float32)
        + jnp.dot(hn2, wc_ref[D:2 * D], preferred_element_type=jnp.float32)
        + jnp.dot(x_ref[...], wc_ref[2 * D:3 * D],
                  preferred_element_type=jnp.float32)
        + bc_ref[...])

  return pl.pallas_call(
      body,
      grid=(N_PAD // _BLK,),
      in_specs=[
          pl.BlockSpec((NC, _BLK, D), lambda i: (0, i, 0)),
          pl.BlockSpec((NC, _BLK, 1), lambda i: (0, i, 0)),
          pl.BlockSpec((_BLK, D), lambda i: (i, 0)),
          pl.BlockSpec((_BLK, D), lambda i: (i, 0)),
          _full((1, D)),
          _full((3 * D, D)),
          _full((1, D)),
      ],
      out_specs=pl.BlockSpec((_BLK, D), lambda i: (i, 0)),
      out_shape=jax.ShapeDtypeStruct((N_PAD, D), jnp.float32),
  )(t2, deg.reshape(NC, N_PAD, 1), hs2, x, b2.reshape(1, D), wc,
    bc.reshape(1, D))


def kernel(feat_full, edge_index, emb_table, W_self1, W_neigh1, b1,
           W_self2, W_neigh2, b2, W_cls, b_cls):
  feat_lt = jnp.concatenate([feat_full, feat_full[:, L - (LT - L):]], axis=1)
  feat_lt = jnp.concatenate(
      [feat_lt, jnp.zeros((N_PAD - N, LT), jnp.int32)], axis=0)
  feat_flat = feat_lt.reshape(-1)

  src = edge_index[0]
  dst = edge_index[1]
  src_p = jnp.concatenate([src, jnp.zeros((E_PAD - E,), jnp.int32)])
  dst_p = jnp.concatenate([dst, jnp.full((E_PAD - E,), N, jnp.int32)])

  x_sent = _maxpool_call(feat_flat, emb_table)
  hs1, z1 = _dense1_call(x_sent, W_self1, W_neigh1, b1)
  t1, deg = _agg_call(z1, src_p, dst_p, with_deg=True)
  hs2, z2 = _dense2_call(t1, deg, hs1, b1, W_self2, W_neigh2, b2)
  (t2,) = _agg_call(z2, src_p, dst_p, with_deg=False)

  wc_pad = jnp.concatenate(
      [W_cls, jnp.zeros((3 * D, D - C), jnp.float32)], axis=1)
  bc_pad = jnp.concatenate([b_cls, jnp.zeros((D - C,), jnp.float32)])
  pred = _cls_call(t2, deg, hs2, x_sent, b2, wc_pad, bc_pad)
  return pred[:N, :C]


# speedup vs baseline: 2.5901x; 1.0723x over previous
import functools

import jax
import jax.numpy as jnp
from jax import lax
from jax.experimental import pallas as pl
from jax.experimental.pallas import tpu as pltpu
from jax.experimental.pallas import tpu_sc as plsc

N = 10000
L = 50
VOCAB = 30000
E = 160000
D = 128
C = 50

NC = 2
NS = 16
NW = NC * NS

N_PAD = 10240
NODES_PER_TILE = N_PAD // NW
LT = 52
E_PAD = 163840
EPT = E_PAD // NW
ECHUNK = 128
NECH = EPT // ECHUNK
STRIPE = N_PAD // NS

_SC_MESH = dict(core_axis_name="c", subcore_axis_name="s")


def _worker_id():
  return lax.axis_index("s") * NC + lax.axis_index("c")


_MP_NBUF = 4


def _maxpool_body(feat_ref, emb_ref, x_ref, idx_v, rv, out_v, *sems):
  w = _worker_id()
  node_base = w * NODES_PER_TILE
  pltpu.sync_copy(
      feat_ref.at[pl.ds(node_base * LT, NODES_PER_TILE * LT)], idx_v)

  n_chunks = NODES_PER_TILE // 2

  def gather_start(j, b):
    pltpu.make_async_copy(
        emb_ref.at[idx_v.at[pl.ds(j * (2 * LT), 2 * LT)]],
        rv.at[b], sems[b]).start()

  def gather_wait(b):
    pltpu.make_async_copy(
        emb_ref.at[idx_v.at[pl.ds(0, 2 * LT)]], rv.at[b], sems[b]).wait()

  for b in range(_MP_NBUF):
    gather_start(b, b)

  @pl.loop(0, n_chunks, step=_MP_NBUF)
  def _chunks(j):
    for b in range(_MP_NBUF):
      jj = j + b
      gather_wait(b)
      for n in range(2):
        row0 = n * LT
        acc = tuple(rv[b, row0, pl.ds(k * 16, 16)] for k in range(8))

        def rbody(it, a, _b=b, _row0=row0):
          for dr in range(7):
            r = _row0 + 1 + it * 7 + dr
            a = tuple(
                jnp.maximum(a[k], rv[_b, r, pl.ds(k * 16, 16)])
                for k in range(8))
          return a

        acc = lax.fori_loop(0, (L - 1) // 7, rbody, acc)
        for k in range(8):
          out_v[jj * 2 + n, pl.ds(k * 16, 16)] = acc[k]

      @pl.when(jj + _MP_NBUF < n_chunks)
      def _refill(_jj=jj, _b=b):
        gather_start(_jj + _MP_NBUF, _b)

  pltpu.sync_copy(out_v, x_ref.at[pl.ds(node_base, NODES_PER_TILE)])


def _maxpool_call(feat_flat, emb_table):
  mesh = plsc.VectorSubcoreMesh(**_SC_MESH)
  f = pl.kernel(
      _maxpool_body,
      out_type=jax.ShapeDtypeStruct((N_PAD, D), jnp.float32),
      mesh=mesh,
      scratch_types=[
          pltpu.VMEM((NODES_PER_TILE * LT,), jnp.int32),
          pltpu.VMEM((_MP_NBUF, 2 * LT, D), jnp.float32),
          pltpu.VMEM((NODES_PER_TILE, D), jnp.float32),
      ] + [pltpu.SemaphoreType.DMA] * _MP_NBUF,
  )
  return f(feat_flat, emb_table)


def _agg_body(with_deg, z_ref, src_ref, dst_ref, t_ref, deg_ref,
              src_v, dst_v, rv, ones_v, zd_v, agg_sh, deg_sh, sem0, sem1):
  c = lax.axis_index("c")
  s = lax.axis_index("s")
  w = s * NC + c
  pltpu.sync_copy(src_ref.at[pl.ds(w * EPT, EPT)], src_v)
  pltpu.sync_copy(dst_ref.at[pl.ds(w * EPT, EPT)], dst_v)

  zeros16 = jnp.zeros((16,), jnp.float32)

  @pl.loop(0, ECHUNK)
  def _zrow(i):
    for k in range(8):
      rv[0, i, pl.ds(k * 16, 16)] = zeros16

  stripe0 = s * STRIPE
  for q in range(STRIPE // ECHUNK):
    pltpu.sync_copy(rv.at[0], agg_sh.at[pl.ds(stripe0 + q * ECHUNK, ECHUNK)])

  if with_deg:
    ones16 = jnp.ones((16,), jnp.float32)
    for k in range(8):
      ones_v[pl.ds(k * 16, 16)] = ones16

    @pl.loop(0, STRIPE // 16)
    def _zdeg(i):
      zd_v[pl.ds(i * 16, 16)] = zeros16

    pltpu.sync_copy(zd_v, deg_sh.at[pl.ds(stripe0, STRIPE)])

  plsc.subcore_barrier()

  sems = (sem0, sem1)

  def gather_start(j, b):
    pltpu.make_async_copy(
        z_ref.at[src_v.at[pl.ds(j * ECHUNK, ECHUNK)]], rv.at[b],
        sems[b]).start()

  def gather_wait(b):
    pltpu.make_async_copy(
        z_ref.at[src_v.at[pl.ds(0, ECHUNK)]], rv.at[b], sems[b]).wait()

  gather_start(0, 0)
  gather_start(1, 1)

  @pl.loop(0, NECH, step=2)
  def _chunks(j):
    for b in range(2):
      jj = j + b
      gather_wait(b)
      dsl = dst_v.at[pl.ds(jj * ECHUNK, ECHUNK)]
      pltpu.sync_copy(rv.at[b], agg_sh.at[dsl], add=True)
      if with_deg:
        pltpu.sync_copy(ones_v, deg_sh.at[dsl], add=True)

      @pl.when(jj + 2 < NECH)
      def _refill(_jj=jj, _b=b):
        gather_start(_jj + 2, _b)

  plsc.subcore_barrier()

  pltpu.sync_copy(agg_sh.at[pl.ds(stripe0, STRIPE)],
                  t_ref.at[c].at[pl.ds(stripe0, STRIPE)])
  if with_deg:
    pltpu.sync_copy(deg_sh.at[pl.ds(stripe0, STRIPE)],
                    deg_ref.at[c].at[pl.ds(stripe0, STRIPE)])


def _agg_call(z, src, dst, with_deg):
  mesh = plsc.VectorSubcoreMesh(**_SC_MESH)
  out_type = [jax.ShapeDtypeStruct((NC, N_PAD, D), jnp.float32)]
  if with_deg:
    out_type.append(jax.ShapeDtypeStruct((NC, N_PAD), jnp.float32))

  def body(z_ref, src_ref, dst_ref, *rest):
    if with_deg:
      t_ref, deg_ref = rest[0], rest[1]
      scr = rest[2:]
    else:
      t_ref, deg_ref = rest[0], None
      scr = rest[1:]
    _agg_body(with_deg, z_ref, src_ref, dst_ref, t_ref, deg_ref, *scr)

  f = pl.kernel(
      body,
      out_type=tuple(out_type),
      mesh=mesh,
      scratch_types=[
          pltpu.VMEM((EPT,), jnp.int32),
          pltpu.VMEM((EPT,), jnp.int32),
          pltpu.VMEM((2, ECHUNK, D), jnp.float32),
          pltpu.VMEM((ECHUNK,), jnp.float32),
          pltpu.VMEM((STRIPE,), jnp.float32),
          pltpu.VMEM_SHARED((N_PAD, D), jnp.float32),
          pltpu.VMEM_SHARED((N_PAD,), jnp.float32),
          pltpu.SemaphoreType.DMA,
          pltpu.SemaphoreType.DMA,
      ],
  )
  return f(z, src, dst)


_BLK = 1280


def _full(shape):
  return pl.BlockSpec(shape, lambda i: tuple(0 for _ in shape))


def _dense1_call(x, ws, wn, b):
  def body(x_ref, ws_ref, wn_ref, b_ref, hs_ref, z_ref):
    xb = x_ref[...]
    bb = b_ref[...]
    hs_ref[...] = jnp.maximum(
        jnp.dot(xb, ws_ref[...], preferred_element_type=jnp.float32) + bb, 0.0)
    z_ref[...] = jnp.dot(xb, wn_ref[...], preferred_element_type=jnp.float32)

  return pl.pallas_call(
      body,
      grid=(N_PAD // _BLK,),
      in_specs=[
          pl.BlockSpec((_BLK, D), lambda i: (i, 0)),
          _full((D, D)),
          _full((D, D)),
          _full((1, D)),
      ],
      out_specs=[
          pl.BlockSpec((_BLK, D), lambda i: (i, 0)),
          pl.BlockSpec((_BLK, D), lambda i: (i, 0)),
      ],
      out_shape=[
          jax.ShapeDtypeStruct((N_PAD, D), jnp.float32),
          jax.ShapeDtypeStruct((N_PAD, D), jnp.float32),
      ],
  )(x, ws, wn, b.reshape(1, D))


def _dense2_call(t1, deg, hs1, b1, ws2, wn2, b2):
  def body(t_ref, deg_ref, hs1_ref, b1_ref, ws2_ref, wn2_ref, b2_ref,
           hs2_ref, z2_ref):
    t = t_ref[0] + t_ref[1]
    dg = deg_ref[0] + deg_ref[1]
    dinv = 1.0 / jnp.maximum(dg, 1.0)
    hn1 = jnp.maximum(t * dinv + b1_ref[...], 0.0)
    hs1b = hs1_ref[...]
    hs2_ref[...] = jnp.maximum(
        jnp.dot(hs1b, ws2_ref[0:D], preferred_element_type=jnp.float32)
        + jnp.dot(hn1, ws2_ref[D:2 * D], preferred_element_type=jnp.float32)
        + b2_ref[...], 0.0)
    z2_ref[...] = (
        jnp.dot(hs1b, wn2_ref[0:D], preferred_element_type=jnp.float32)
        + jnp.dot(hn1, wn2_ref[D:2 * D], preferred_element_type=jnp.float32))

  return pl.pallas_call(
      body,
      grid=(N_PAD // _BLK,),
      in_specs=[
          pl.BlockSpec((NC, _BLK, D), lambda i: (0, i, 0)),
          pl.BlockSpec((NC, _BLK, 1), lambda i: (0, i, 0)),
          pl.BlockSpec((_BLK, D), lambda i: (i, 0)),
          _full((1, D)),
          _full((2 * D, D)),
          _full((2 * D, D)),
          _full((1, D)),
      ],
      out_specs=[
          pl.BlockSpec((_BLK, D), lambda i: (i, 0)),
          pl.BlockSpec((_BLK, D), lambda i: (i, 0)),
      ],
      out_shape=[
          jax.ShapeDtypeStruct((N_PAD, D), jnp.float32),
          jax.ShapeDtypeStruct((N_PAD, D), jnp.float32),
      ],
  )(t1, deg.reshape(NC, N_PAD, 1), hs1, b1.reshape(1, D), ws2, wn2,
    b2.reshape(1, D))


def _cls_call(t2, deg, hs2, x, b2, wc, bc):
  def body(t_ref, deg_ref, hs2_ref, x_ref, b2_ref, wc_ref, bc_ref, out_ref):
    t = t_ref[0] + t_ref[1]
    dg = deg_ref[0] + deg_ref[1]
    dinv = 1.0 / jnp.maximum(dg, 1.0)
    hn2 = jnp.maximum(t * dinv + b2_ref[...], 0.0)
    out_ref[...] = (
        jnp.dot(hs2_ref[...], wc_ref[0:D], preferred_element_type=jnp.float32)
        + jnp.dot(hn2, wc_ref[D:2 * D], preferred_element_type=jnp.float32)
        + jnp.dot(x_ref[...], wc_ref[2 * D:3 * D],
                  preferred_element_type=jnp.float32)
        + bc_ref[...])

  return pl.pallas_call(
      body,
      grid=(N_PAD // _BLK,),
      in_specs=[
          pl.BlockSpec((NC, _BLK, D), lambda i: (0, i, 0)),
          pl.BlockSpec((NC, _BLK, 1), lambda i: (0, i, 0)),
          pl.BlockSpec((_BLK, D), lambda i: (i, 0)),
          pl.BlockSpec((_BLK, D), lambda i: (i, 0)),
          _full((1, D)),
          _full((3 * D, D)),
          _full((1, D)),
      ],
      out_specs=pl.BlockSpec((_BLK, D), lambda i: (i, 0)),
      out_shape=jax.ShapeDtypeStruct((N_PAD, D), jnp.float32),
  )(t2, deg.reshape(NC, N_PAD, 1), hs2, x, b2.reshape(1, D), wc,
    bc.reshape(1, D))


def kernel(feat_full, edge_index, emb_table, W_self1, W_neigh1, b1,
           W_self2, W_neigh2, b2, W_cls, b_cls):
  feat_lt = jnp.concatenate([feat_full, feat_full[:, L - (LT - L):]], axis=1)
  feat_lt = jnp.concatenate(
      [feat_lt, jnp.zeros((N_PAD - N, LT), jnp.int32)], axis=0)
  feat_flat = feat_lt.reshape(-1)

  src = edge_index[0]
  dst = edge_index[1]
  src_p = jnp.concatenate([src, jnp.zeros((E_PAD - E,), jnp.int32)])
  dst_p = jnp.concatenate([dst, jnp.full((E_PAD - E,), N, jnp.int32)])

  x_sent = _maxpool_call(feat_flat, emb_table)
  hs1, z1 = _dense1_call(x_sent, W_self1, W_neigh1, b1)
  t1, deg = _agg_call(z1, src_p, dst_p, with_deg=True)
  hs2, z2 = _dense2_call(t1, deg, hs1, b1, W_self2, W_neigh2, b2)
  (t2,) = _agg_call(z2, src_p, dst_p, with_deg=False)

  wc_pad = jnp.concatenate(
      [W_cls, jnp.zeros((3 * D, D - C), jnp.float32)], axis=1)
  bc_pad = jnp.concatenate([b_cls, jnp.zeros((D - C,), jnp.float32)])
  pred = _cls_call(t2, deg, hs2, x_sent, b2, wc_pad, bc_pad)
  return pred[:N, :C]
